# baseline (speedup 1.0000x reference)
import jax
import jax.numpy as jnp
from jax.experimental import pallas as pl

BATCH = 64
DIM_CODES = 128
DICT_SIZE = 8192
EMBED_DIM = 32


def _vq_kernel(mu_ref, dict_ref, z_ref, oh_ref):
    mu = mu_ref[0]
    d = dict_ref[0]
    cross = jax.lax.dot_general(
        mu, d, (((1,), (1,)), ((), ())), preferred_element_type=jnp.float32)
    mu_sq = jnp.sum(mu * mu, axis=1)
    d_sq = jnp.sum(d * d, axis=1)
    dist = mu_sq[:, None] + d_sq[None, :] - 2.0 * cross
    idx = jnp.argmin(dist, axis=1)
    iota = jax.lax.broadcasted_iota(jnp.int32, (BATCH, DICT_SIZE), 1)
    oh = (iota == idx[:, None]).astype(jnp.float32)
    oh_ref[:, 0, 0, :] = oh
    z_ref[0] = jax.lax.dot_general(
        oh, d, (((1,), (0,)), ((), ())), preferred_element_type=jnp.float32)


def kernel(mu, dictionary):
    B, C, K, E = BATCH, DIM_CODES, DICT_SIZE, EMBED_DIM
    mu_t = mu.reshape(B, C, E).transpose(1, 0, 2)
    z_t, one_hot = pl.pallas_call(
        _vq_kernel,
        grid=(C,),
        in_specs=[
            pl.BlockSpec((1, B, E), lambda c: (c, 0, 0)),
            pl.BlockSpec((1, K, E), lambda c: (c, 0, 0)),
        ],
        out_specs=[
            pl.BlockSpec((1, B, E), lambda c: (c, 0, 0)),
            pl.BlockSpec((B, 1, 1, K), lambda c: (0, c, 0, 0)),
        ],
        out_shape=[
            jax.ShapeDtypeStruct((C, B, E), jnp.float32),
            jax.ShapeDtypeStruct((B, C, 1, K), jnp.float32),
        ],
    )(mu_t, dictionary)
    z = z_t.transpose(1, 0, 2).reshape(B, C * E)
    return (z, z, one_hot.reshape(B, C, K))
